# baseline (speedup 1.0000x reference)
import functools

import jax
import jax.numpy as jnp
from jax.experimental import pallas as pl
from jax.experimental.pallas import tpu as pltpu

BLOCK_M = 16384
N_QUEUES = 8


def _body(x_ref, w_ref, b_ref, o_hbm, scratch, sems, *, block_m, n_blocks):
    i = pl.program_id(0)
    sub = block_m // N_QUEUES

    o = (
        jnp.dot(x_ref[...], w_ref[...], preferred_element_type=jnp.float32)
        + b_ref[...]
    )

    buf = jax.lax.rem(i, 2)

    def _copy(step, slot, k):
        return pltpu.make_async_copy(
            scratch.at[slot, pl.ds(k * sub, sub), :],
            o_hbm.at[pl.ds(step * block_m + k * sub, sub), :],
            sems.at[slot, k],
        )

    @pl.when(i >= 2)
    def _():
        for k in range(N_QUEUES):
            _copy(i - 2, buf, k).wait()

    scratch[buf] = o
    for k in range(N_QUEUES):
        _copy(i, buf, k).start()

    @pl.when(i == n_blocks - 1)
    def _():
        @pl.when(i >= 1)
        def _():
            for k in range(N_QUEUES):
                _copy(i - 1, 1 - buf, k).wait()

        for k in range(N_QUEUES):
            _copy(i, buf, k).wait()


@functools.partial(jax.jit, static_argnames=())
def kernel(features, indices, W, b):
    del indices
    m, c_in = features.shape
    c_out = W.shape[1]
    block_m = min(BLOCK_M, m)
    n_blocks = pl.cdiv(m, block_m)
    body = functools.partial(_body, block_m=block_m, n_blocks=n_blocks)
    return pl.pallas_call(
        body,
        grid=(n_blocks,),
        in_specs=[
            pl.BlockSpec((block_m, c_in), lambda i: (i, 0)),
            pl.BlockSpec((c_in, c_out), lambda i: (0, 0)),
            pl.BlockSpec((1, c_out), lambda i: (0, 0)),
        ],
        out_specs=pl.BlockSpec(memory_space=pltpu.MemorySpace.HBM),
        out_shape=jax.ShapeDtypeStruct((m, c_out), jnp.float32),
        scratch_shapes=[
            pltpu.VMEM((2, block_m, c_out), jnp.float32),
            pltpu.SemaphoreType.DMA((2, N_QUEUES)),
        ],
    )(features, W, b.reshape(1, c_out))
